# baseline (speedup 1.0000x reference)
import functools

import jax
import jax.numpy as jnp
from jax import lax
from jax.experimental import pallas as pl
from jax.experimental.pallas import tpu as pltpu
from jax.experimental.pallas import tpu_sc as plsc

B, C, H, W = 2, 384, 224, 224
OH, OW, K = 112, 112, 4
NC, NS, L = 2, 16, 16

NPLANES = B * C
PLANE = H * W
NPIX = OH * OW
PLANES_PER_CORE = NPLANES // NC
PIX_PER_TILE = NPIX // NS
GROUPS = PIX_PER_TILE // L
NJ = 4 * K


def _sc_body(x_hbm, sm_hbm, out_hbm, idx_v, w_v, out_v):
    c = lax.axis_index("c")
    s = lax.axis_index("s")

    def precompute(sm_v):
        pltpu.sync_copy(sm_hbm.at[pl.ds(s * (PIX_PER_TILE * K * 2),
                                        PIX_PER_TILE * K * 2)], sm_v)
        lane = lax.iota(jnp.int32, L)

        def g_body(g, carry):
            p16 = g * L + lane
            for k in range(K):
                off = p16 * (2 * K) + (2 * k)
                xs = plsc.load_gather(sm_v, [off])
                ys = plsc.load_gather(sm_v, [off + 1])
                x0i = xs.astype(jnp.int32)
                y0i = ys.astype(jnp.int32)
                wx1 = xs - x0i.astype(jnp.float32)
                wx0 = 1.0 - wx1
                wy1 = ys - y0i.astype(jnp.float32)
                wy0 = 1.0 - wy1
                x0c = jnp.minimum(jnp.maximum(x0i, 0), W - 1)
                x1c = jnp.minimum(x0c + 1, W - 1)
                y0c = jnp.minimum(jnp.maximum(y0i, 0), H - 1)
                y1c = jnp.minimum(y0c + 1, H - 1)
                r0 = y0c * W
                r1 = y1c * W
                scale = 1.0 / K
                sl = pl.ds(g * L, L)
                idx_v[4 * k + 0, sl] = r0 + x0c
                w_v[4 * k + 0, sl] = wy0 * wx0 * scale
                idx_v[4 * k + 1, sl] = r0 + x1c
                w_v[4 * k + 1, sl] = wy0 * wx1 * scale
                idx_v[4 * k + 2, sl] = r1 + x0c
                w_v[4 * k + 2, sl] = wy1 * wx0 * scale
                idx_v[4 * k + 3, sl] = r1 + x1c
                w_v[4 * k + 3, sl] = wy1 * wx1 * scale
            return carry

        lax.fori_loop(0, GROUPS, g_body, 0)

    def main(plane_v):
        def plane_body(i, carry):
            plane = c * PLANES_PER_CORE + i
            pltpu.sync_copy(x_hbm.at[pl.ds(plane * PLANE, PLANE)], plane_v)

            def g_body(g, carry2):
                sl = pl.ds(g * L, L)
                acc = jnp.zeros((L,), jnp.float32)
                for j in range(NJ):
                    iv = idx_v[j, sl]
                    wv = w_v[j, sl]
                    acc = acc + wv * plsc.load_gather(plane_v, [iv])
                out_v[sl] = acc
                return carry2

            lax.fori_loop(0, GROUPS, g_body, 0)
            pltpu.sync_copy(out_v,
                            out_hbm.at[pl.ds(plane * NPIX + s * PIX_PER_TILE,
                                             PIX_PER_TILE)])
            return carry

        lax.fori_loop(0, PLANES_PER_CORE, plane_body, 0)

    pl.run_scoped(precompute, pltpu.VMEM((PIX_PER_TILE * K * 2,), jnp.float32))
    pl.run_scoped(main, pltpu.VMEM((PLANE,), jnp.float32))


@jax.jit
def kernel(x, sample_map):
    x2 = x.reshape(NPLANES * PLANE)
    smf = sample_map.reshape(-1)
    sc = pl.kernel(
        _sc_body,
        out_type=jax.ShapeDtypeStruct((NPLANES * NPIX,), jnp.float32),
        mesh=plsc.VectorSubcoreMesh(core_axis_name="c", subcore_axis_name="s",
                                    num_cores=NC, num_subcores=NS),
        scratch_types=[
            pltpu.VMEM((NJ, PIX_PER_TILE), jnp.int32),
            pltpu.VMEM((NJ, PIX_PER_TILE), jnp.float32),
            pltpu.VMEM((PIX_PER_TILE,), jnp.float32),
        ],
        compiler_params=pltpu.CompilerParams(needs_layout_passes=False),
    )
    out = sc(x2, smf)
    return out.reshape(B, C, OH, OW)
